# baseline (speedup 1.0000x reference)
import jax
import jax.numpy as jnp
from jax import lax
from jax.experimental import pallas as pl
from jax.experimental.pallas import tpu as pltpu

S = 200
GN = 4096
D = 128
N_COMMANDS = 7
GROUP_VOCAB = 52
VOCAB_PAD = 64
ROWS = 5


def _body(cmd_ref, grp_ref, args_hbm, w1_ref, w2_ref, b_ref, pos_ref, out_ref,
          abuf, sem):
    step = pl.program_id(0)
    nsteps = pl.num_programs(0)

    def start(step_idx, slot):
        pltpu.make_async_copy(
            args_hbm.at[pl.ds(step_idx * ROWS, ROWS)], abuf.at[slot], sem.at[slot],
        ).start()

    def wait(slot):
        pltpu.make_async_copy(
            args_hbm.at[pl.ds(0, ROWS)], abuf.at[slot], sem.at[slot],
        ).wait()

    @pl.when(step == 0)
    def _():
        start(0, 0)

    @pl.when(step + 1 < nsteps)
    def _():
        start(step + 1, (step + 1) % 2)

    slot = step % 2
    wait(slot)

    iota = lax.broadcasted_iota(jnp.int32, (VOCAB_PAD, 1), 0)
    for r in range(ROWS):
        c = cmd_ref[r]
        g = grp_ref[r]
        oh_t = (iota == c).astype(jnp.float32) + (iota == g + N_COMMANDS).astype(jnp.float32)
        acc = lax.dot_general(
            oh_t, w1_ref[...], (((0,), (0,)), ((), ())),
            preferred_element_type=jnp.float32,
        )
        acc = acc + jnp.dot(abuf[slot, r], w2_ref[...],
                            preferred_element_type=jnp.float32)
        pb = pos_ref[r] + b_ref[...]
        out_ref[r] = acc + pb


def kernel(commands, args, groups, command_embed, W_fcn, b_fcn, group_embed, pos_embed):
    w1 = jnp.concatenate(
        [command_embed, group_embed,
         jnp.zeros((VOCAB_PAD - N_COMMANDS - GROUP_VOCAB, D), jnp.float32)], axis=0)
    w2 = W_fcn.T
    b2 = b_fcn.reshape(1, D)
    cmd3 = commands.reshape(S, 1, GN).astype(jnp.int32)
    grp3 = groups.reshape(S, 1, GN).astype(jnp.int32)
    pos3 = pos_embed.reshape(-1, 1, D)
    na = args.shape[-1]

    grid = (S // ROWS,)
    out = pl.pallas_call(
        _body,
        grid=grid,
        in_specs=[
            pl.BlockSpec((ROWS, 1, GN), lambda s: (s, 0, 0)),
            pl.BlockSpec((ROWS, 1, GN), lambda s: (s, 0, 0)),
            pl.BlockSpec(memory_space=pl.ANY),
            pl.BlockSpec((VOCAB_PAD, D), lambda s: (0, 0)),
            pl.BlockSpec((W_fcn.shape[1], D), lambda s: (0, 0)),
            pl.BlockSpec((1, D), lambda s: (0, 0)),
            pl.BlockSpec((ROWS, 1, D), lambda s: (s, 0, 0)),
        ],
        out_specs=pl.BlockSpec((ROWS, GN, D), lambda s: (s, 0, 0)),
        out_shape=jax.ShapeDtypeStruct((S, GN, D), jnp.float32),
        scratch_shapes=[
            pltpu.VMEM((2, ROWS, GN, na), jnp.float32),
            pltpu.SemaphoreType.DMA((2,)),
        ],
    )(cmd3, grp3, args, w1, w2, b2, pos3)
    return out
